# Initial kernel scaffold; baseline (speedup 1.0000x reference)
import jax
import jax.numpy as jnp
from jax.experimental import pallas as pl


def kernel(x, edge_index, batch, Wg1, bg1, g1, b1, Wg2, bg2, g2, b2, Wfc, bfc, Wfc1, bfc1):
    raise NotImplementedError("write your pallas kernel here")


# speedup vs baseline: 6.6798x; 6.6798x over previous
import functools

import jax
import jax.numpy as jnp
from jax.experimental import pallas as pl
from jax.experimental.pallas import tpu as pltpu

N = 10000
G = 64
D_IN = 256
HIDDEN = 256
NUM_CLASSES = 10

BR = 2000
NB = N // BR
EPS = 1e-5


def _fused_kernel(x_ref, batch_ref, w1_ref, g1_ref, b1_ref, w2_ref, g2_ref,
                  b2_ref, wfc_ref, bfc_ref, wfc1_ref, bfc1_ref, out_ref,
                  h1_ref, h2_ref, sum1_ref, sq1_ref, sum2_ref, sq2_ref,
                  pooled_ref, cnt_ref):
    pid = pl.program_id(0)
    fN = jnp.float32(N)

    @pl.when(pid < NB)
    def _phase_a():
        j = pid
        h = jnp.dot(x_ref[...], w1_ref[...], preferred_element_type=jnp.float32)
        h1_ref[pl.ds(j * BR, BR), :] = h
        s = jnp.sum(h, axis=0, keepdims=True)
        q = jnp.sum(h * h, axis=0, keepdims=True)

        @pl.when(j == 0)
        def _():
            sum1_ref[...] = s
            sq1_ref[...] = q

        @pl.when(j > 0)
        def _():
            sum1_ref[...] += s
            sq1_ref[...] += q

    @pl.when((pid >= NB) & (pid < 2 * NB))
    def _phase_b():
        j = pid - NB
        mu = sum1_ref[...] / fN
        var = sq1_ref[...] / fN - mu * mu
        a = g1_ref[...] * jax.lax.rsqrt(var + EPS)
        c = b1_ref[...] - mu * a
        hb = h1_ref[pl.ds(j * BR, BR), :]
        hn = jnp.maximum(hb * a + c, 0.0)
        h2 = jnp.dot(hn, w2_ref[...], preferred_element_type=jnp.float32)
        h2_ref[pl.ds(j * BR, BR), :] = h2
        s = jnp.sum(h2, axis=0, keepdims=True)
        q = jnp.sum(h2 * h2, axis=0, keepdims=True)

        @pl.when(j == 0)
        def _():
            sum2_ref[...] = s
            sq2_ref[...] = q

        @pl.when(j > 0)
        def _():
            sum2_ref[...] += s
            sq2_ref[...] += q

    @pl.when((pid >= 2 * NB) & (pid < 3 * NB))
    def _phase_c():
        j = pid - 2 * NB
        mu = sum2_ref[...] / fN
        var = sq2_ref[...] / fN - mu * mu
        a = g2_ref[...] * jax.lax.rsqrt(var + EPS)
        c = b2_ref[...] - mu * a
        hb = h2_ref[pl.ds(j * BR, BR), :]
        hn = jnp.maximum(hb * a + c, 0.0)
        oh = (batch_ref[...] ==
              jax.lax.broadcasted_iota(jnp.int32, (BR, G), 1)
              ).astype(jnp.float32)
        dn = (((0,), (0,)), ((), ()))
        pb = jax.lax.dot_general(oh, hn, dn,
                                 preferred_element_type=jnp.float32)
        cb = jax.lax.dot_general(oh, jnp.ones((BR, 8), jnp.float32), dn,
                                 preferred_element_type=jnp.float32)

        @pl.when(j == 0)
        def _():
            pooled_ref[...] = pb
            cnt_ref[...] = cb

        @pl.when(j > 0)
        def _():
            pooled_ref[...] += pb
            cnt_ref[...] += cb

    @pl.when(pid == 3 * NB)
    def _phase_d():
        cnt = jnp.maximum(cnt_ref[:, 0:1], 1.0)
        pooled = pooled_ref[...] / cnt
        h3 = jnp.maximum(
            jnp.dot(pooled, wfc_ref[...], preferred_element_type=jnp.float32)
            + bfc_ref[...], 0.0)
        logits = jnp.dot(h3, wfc1_ref[...],
                         preferred_element_type=jnp.float32) + bfc1_ref[...]
        m = jnp.max(logits, axis=-1, keepdims=True)
        sh = logits - m
        lse = jnp.log(jnp.sum(jnp.exp(sh), axis=-1, keepdims=True))
        out_ref[...] = sh - lse


@functools.partial(jax.jit, static_argnames=("interpret",))
def _run(x, batch, Wg1, g1, b1, Wg2, g2, b2, Wfc, bfc, Wfc1, bfc1,
         interpret=False):
    batch2d = batch.reshape(N, 1)
    grid = (3 * NB + 1,)
    row = lambda r: (r, 0)
    const = lambda i: (0, 0)
    x_map = lambda i: row(jnp.where(i < NB, i, NB - 1))
    b_map = lambda i: row(jnp.clip(i - 2 * NB, 0, NB - 1))
    in_specs = [
        pl.BlockSpec((BR, D_IN), x_map),
        pl.BlockSpec((BR, 1), b_map),
        pl.BlockSpec((D_IN, HIDDEN), const),
        pl.BlockSpec((1, HIDDEN), const),
        pl.BlockSpec((1, HIDDEN), const),
        pl.BlockSpec((HIDDEN, HIDDEN), const),
        pl.BlockSpec((1, HIDDEN), const),
        pl.BlockSpec((1, HIDDEN), const),
        pl.BlockSpec((HIDDEN, HIDDEN), const),
        pl.BlockSpec((1, HIDDEN), const),
        pl.BlockSpec((HIDDEN, NUM_CLASSES), const),
        pl.BlockSpec((1, NUM_CLASSES), const),
    ]
    out = pl.pallas_call(
        _fused_kernel,
        grid=grid,
        in_specs=in_specs,
        out_specs=pl.BlockSpec((G, NUM_CLASSES), const),
        out_shape=jax.ShapeDtypeStruct((G, NUM_CLASSES), jnp.float32),
        scratch_shapes=[
            pltpu.VMEM((N, HIDDEN), jnp.float32),
            pltpu.VMEM((N, HIDDEN), jnp.float32),
            pltpu.VMEM((1, HIDDEN), jnp.float32),
            pltpu.VMEM((1, HIDDEN), jnp.float32),
            pltpu.VMEM((1, HIDDEN), jnp.float32),
            pltpu.VMEM((1, HIDDEN), jnp.float32),
            pltpu.VMEM((G, HIDDEN), jnp.float32),
            pltpu.VMEM((G, 8), jnp.float32),
        ],
        interpret=interpret,
    )(x, batch2d, Wg1, g1.reshape(1, HIDDEN), b1.reshape(1, HIDDEN),
      Wg2, g2.reshape(1, HIDDEN), b2.reshape(1, HIDDEN),
      Wfc, bfc.reshape(1, HIDDEN), Wfc1, bfc1.reshape(1, NUM_CLASSES))
    return out


def kernel(x, edge_index, batch, Wg1, bg1, g1, b1, Wg2, bg2, g2, b2,
           Wfc, bfc, Wfc1, bfc1):
    del edge_index, bg1, bg2
    return _run(x, batch, Wg1, g1, b1, Wg2, g2, b2, Wfc, bfc, Wfc1, bfc1)
